# baseline (speedup 1.0000x reference)
import functools

import jax
import jax.numpy as jnp
from jax import lax
from jax.experimental import pallas as pl
from jax.experimental.pallas import tpu as pltpu
from jax.experimental.pallas import tpu_sc as plsc
from jax.experimental.layout import Layout, with_layout_constraint

D_MODEL = 64
SCALE = 8.0
_L = 16
_NC = 2
_NS = 16
_NW = _NC * _NS
_TR = 8
_TC = 128


@functools.lru_cache(maxsize=None)
def _make_kernel(R, C, V):
    rpw = R // _NW
    cb_n = C // _TR
    db_n = D_MODEL // _TR
    mesh = plsc.VectorSubcoreMesh(core_axis_name="c", subcore_axis_name="s")

    @functools.partial(
        pl.kernel,
        mesh=mesh,
        out_type=jax.ShapeDtypeStruct(
            (C, db_n, _NW, _TR, rpw), jnp.float32
        ),
        scratch_types=[
            pltpu.VMEM((cb_n, 1, _TR, rpw), jnp.int32),
            pltpu.VMEM((2, rpw, D_MODEL), jnp.float32),
            pltpu.VMEM((2, 1, db_n, 1, _TR, rpw), jnp.float32),
            pltpu.SemaphoreType.DMA,
            pltpu.SemaphoreType.DMA,
            pltpu.SemaphoreType.DMA,
            pltpu.SemaphoreType.DMA,
        ],
        compiler_params=pltpu.CompilerParams(
            use_tc_tiling_on_sc=False, needs_layout_passes=False
        ),
    )
    def k(xq_hbm, lut_hbm, out_hbm, idx_v, rows_v, trows_v, g0, g1, s0, s1):
        wid = lax.axis_index("s") * _NC + lax.axis_index("c")
        pltpu.sync_copy(xq_hbm.at[:, pl.ds(wid, 1)], idx_v)
        gsem = (g0, g1)
        ssem = (s0, s1)

        def gather_copy(c, b):
            idx_ref = idx_v.at[c // _TR, 0, c % _TR]
            return pltpu.make_async_copy(
                lut_hbm.at[idx_ref], rows_v.at[b], gsem[b]
            )

        def store_copy(c, b):
            return pltpu.make_async_copy(
                trows_v.at[b],
                out_hbm.at[pl.ds(c, 1), pl.ds(0, db_n), pl.ds(wid, 1)],
                ssem[b],
            )

        riota = lax.iota(jnp.int32, _L)

        gather_copy(0, 0).start()
        gather_copy(1, 1).start()

        def outer(o, carry):
            for b in range(2):
                c = o * 2 + b
                gather_copy(c, b).wait()

                @pl.when(c >= 2)
                def _():
                    store_copy(c - 2, b).wait()

                src = rows_v.at[b]

                def tr_body(db, carry2):
                    for di in range(_TR):
                        dvec = jnp.full((_L,), db * _TR + di, jnp.int32)
                        for g in range(rpw // _L):
                            rvec = riota + (g * _L)
                            vals = plsc.load_gather(src, [rvec, dvec])
                            trows_v[
                                b, 0, db, 0, di, pl.ds(g * _L, _L)
                            ] = vals * SCALE
                    return carry2

                lax.fori_loop(0, db_n, tr_body, 0)
                store_copy(c, b).start()

                @pl.when(c + 2 < C)
                def _():
                    gather_copy(c + 2, b).start()

            return carry

        lax.fori_loop(0, C // 2, outer, 0)
        store_copy(C - 2, 0).wait()
        store_copy(C - 1, 1).wait()

    return k


def kernel(x, lut):
    R, C = x.shape
    V = lut.shape[0]
    rpw = R // _NW
    cb_n = C // _TR
    db_n = D_MODEL // _TR

    x2 = x.astype(jnp.int32).T
    x2 = with_layout_constraint(
        x2, Layout(major_to_minor=(0, 1), tiling=((_TR, _TC),))
    )
    x3 = x2.reshape(cb_n, _TR, _NW, rpw)
    x3 = with_layout_constraint(
        x3, Layout(major_to_minor=(0, 2, 1, 3), tiling=())
    )
    x4 = x3.transpose(0, 2, 1, 3)
    x4 = with_layout_constraint(
        x4, Layout(major_to_minor=(0, 1, 2, 3), tiling=())
    )

    o5 = _make_kernel(R, C, V)(x4, lut)

    t = o5.transpose(2, 4, 0, 1, 3)
    t = with_layout_constraint(
        t, Layout(major_to_minor=(2, 3, 0, 4, 1), tiling=())
    )
    return t.reshape(R, C, D_MODEL)


# speedup vs baseline: 1.7815x; 1.7815x over previous
import functools

import jax
import jax.numpy as jnp
from jax import lax
from jax.experimental import pallas as pl
from jax.experimental.pallas import tpu as pltpu
from jax.experimental.pallas import tpu_sc as plsc
from jax.experimental.layout import Layout, with_layout_constraint

D_MODEL = 64
SCALE = 8.0
_L = 16
_NC = 2
_NS = 16
_NW = _NC * _NS
_TR = 8
_TC = 128


@functools.lru_cache(maxsize=None)
def _make_kernel(R, C, V):
    rpw = R // _NW
    cb_n = C // _TR
    db_n = D_MODEL // _TR
    mesh = plsc.VectorSubcoreMesh(core_axis_name="c", subcore_axis_name="s")

    @functools.partial(
        pl.kernel,
        mesh=mesh,
        out_type=jax.ShapeDtypeStruct(
            (C, db_n, _NW, _TR, rpw), jnp.float32
        ),
        scratch_types=[
            pltpu.VMEM((cb_n, 1, _TR, rpw), jnp.int32),
            pltpu.VMEM((2, rpw, D_MODEL), jnp.float32),
            pltpu.VMEM((2, 1, db_n, 1, _TR, rpw + 1), jnp.float32),
            pltpu.SemaphoreType.DMA,
            pltpu.SemaphoreType.DMA,
            pltpu.SemaphoreType.DMA,
            pltpu.SemaphoreType.DMA,
        ],
        compiler_params=pltpu.CompilerParams(
            use_tc_tiling_on_sc=False, needs_layout_passes=False
        ),
    )
    def k(xq_hbm, lut_hbm, out_hbm, idx_v, rows_v, trows_v, g0, g1, s0, s1):
        wid = lax.axis_index("s") * _NC + lax.axis_index("c")
        pltpu.sync_copy(xq_hbm.at[:, pl.ds(wid, 1)], idx_v)
        gsem = (g0, g1)
        ssem = (s0, s1)

        def gather_copy(c, b):
            idx_ref = idx_v.at[c // _TR, 0, c % _TR]
            return pltpu.make_async_copy(
                lut_hbm.at[idx_ref], rows_v.at[b], gsem[b]
            )

        def store_copy(c, b):
            src = trows_v.at[
                b,
                pl.ds(0, 1),
                pl.ds(0, db_n),
                pl.ds(0, 1),
                pl.ds(0, _TR),
                pl.ds(0, rpw),
            ]
            return pltpu.make_async_copy(
                src,
                out_hbm.at[pl.ds(c, 1), pl.ds(0, db_n), pl.ds(wid, 1)],
                ssem[b],
            )

        riota = lax.iota(jnp.int32, _L)
        zvec = jnp.zeros((_L,), jnp.int32)
        dbvecs = [(riota + j * _L) // _TR for j in range(D_MODEL // _L)]
        divecs = [(riota + j * _L) % _TR for j in range(D_MODEL // _L)]

        gather_copy(0, 0).start()
        gather_copy(1, 1).start()

        def outer(o, carry):
            for b in range(2):
                c = o * 2 + b
                gather_copy(c, b).wait()

                @pl.when(c >= 2)
                def _():
                    store_copy(c - 2, b).wait()

                tref = trows_v.at[b]

                def tr_body(r0, carry2):
                    for ru in range(4):
                        r = r0 * 4 + ru
                        rsplat = jnp.full((_L,), 0, jnp.int32) + r
                        for j in range(D_MODEL // _L):
                            vals = rows_v[b, r, pl.ds(j * _L, _L)] * SCALE
                            plsc.store_scatter(
                                tref,
                                [zvec, dbvecs[j], zvec, divecs[j], rsplat],
                                vals,
                            )
                    return carry2

                lax.fori_loop(0, rpw // 4, tr_body, 0)
                store_copy(c, b).start()

                @pl.when(c + 2 < C)
                def _():
                    gather_copy(c + 2, b).start()

            return carry

        lax.fori_loop(0, C // 2, outer, 0)
        store_copy(C - 2, 0).wait()
        store_copy(C - 1, 1).wait()

    return k


def kernel(x, lut):
    R, C = x.shape
    V = lut.shape[0]
    rpw = R // _NW
    cb_n = C // _TR
    db_n = D_MODEL // _TR

    x2 = x.astype(jnp.int32).T
    x2 = with_layout_constraint(
        x2, Layout(major_to_minor=(0, 1), tiling=((_TR, _TC),))
    )
    x3 = x2.reshape(cb_n, _TR, _NW, rpw)
    x3 = with_layout_constraint(
        x3, Layout(major_to_minor=(0, 2, 1, 3), tiling=())
    )
    x4 = x3.transpose(0, 2, 1, 3)
    x4 = with_layout_constraint(
        x4, Layout(major_to_minor=(0, 1, 2, 3), tiling=())
    )

    o5 = _make_kernel(R, C, V)(x4, lut)

    t = o5.transpose(2, 4, 0, 1, 3)
    t = with_layout_constraint(
        t, Layout(major_to_minor=(2, 3, 0, 4, 1), tiling=())
    )
    return t.reshape(R, C, D_MODEL)


# speedup vs baseline: 2.3206x; 1.3026x over previous
import functools

import jax
import jax.numpy as jnp
from jax import lax
from jax.experimental import pallas as pl
from jax.experimental.pallas import tpu as pltpu
from jax.experimental.pallas import tpu_sc as plsc
from jax.experimental.layout import Layout, with_layout_constraint

D_MODEL = 64
SCALE = 8.0
_L = 16
_NC = 2
_NS = 16
_NW = _NC * _NS
_TR = 8
_TC = 128


@functools.lru_cache(maxsize=None)
def _make_kernel(R, C, V):
    rpw = R // _NW
    cb_n = C // _TR
    db_n = D_MODEL // _TR
    mesh = plsc.VectorSubcoreMesh(core_axis_name="c", subcore_axis_name="s")

    @functools.partial(
        pl.kernel,
        mesh=mesh,
        out_type=jax.ShapeDtypeStruct(
            (C, db_n, _NW, _TR, rpw), jnp.float32
        ),
        scratch_types=[
            pltpu.VMEM((cb_n, 1, _TR, rpw), jnp.int32),
            pltpu.VMEM((2, rpw, D_MODEL), jnp.float32),
            pltpu.VMEM((2, 1, db_n, 1, _TR, rpw + 1), jnp.float32),
            pltpu.SemaphoreType.DMA,
            pltpu.SemaphoreType.DMA,
            pltpu.SemaphoreType.DMA,
            pltpu.SemaphoreType.DMA,
        ],
        compiler_params=pltpu.CompilerParams(
            use_tc_tiling_on_sc=False, needs_layout_passes=False
        ),
    )
    def k(xq_hbm, lut_hbm, out_hbm, idx_v, rows_v, trows_v, g0, g1, s0, s1):
        wid = lax.axis_index("s") * _NC + lax.axis_index("c")
        pltpu.sync_copy(xq_hbm.at[:, pl.ds(wid, 1)], idx_v)
        gsem = (g0, g1)
        ssem = (s0, s1)

        def gather_copy(c, b):
            idx_ref = idx_v.at[c // _TR, 0, c % _TR]
            return pltpu.make_async_copy(
                lut_hbm.at[idx_ref], rows_v.at[b], gsem[b]
            )

        def store_copy(c, b):
            src = trows_v.at[
                b,
                pl.ds(0, 1),
                pl.ds(0, db_n),
                pl.ds(0, 1),
                pl.ds(0, _TR),
                pl.ds(0, rpw),
            ]
            return pltpu.make_async_copy(
                src,
                out_hbm.at[pl.ds(c, 1), pl.ds(0, db_n), pl.ds(wid, 1)],
                ssem[b],
            )

        riota = lax.iota(jnp.int32, _L)
        zvec = jnp.zeros((_L,), jnp.int32)
        dbvecs = [(riota + j * _L) // _TR for j in range(D_MODEL // _L)]
        divecs = [(riota + j * _L) % _TR for j in range(D_MODEL // _L)]

        gather_copy(0, 0).start()
        gather_copy(1, 1).start()

        def outer(o, carry):
            for b in range(2):
                c = o * 2 + b
                gather_copy(c, b).wait()

                @pl.when(c >= 2)
                def _():
                    store_copy(c - 2, b).wait()

                tref = trows_v.at[b]

                def tr_body(r0, carry2):
                    for ru in range(4):
                        r = r0 * 4 + ru
                        rsplat = jnp.full((_L,), 0, jnp.int32) + r
                        for j in range(D_MODEL // _L):
                            vals = rows_v[b, r, pl.ds(j * _L, _L)] * SCALE
                            plsc.store_scatter(
                                tref,
                                [zvec, dbvecs[j], zvec, divecs[j], rsplat],
                                vals,
                            )
                    return carry2

                lax.fori_loop(0, rpw // 4, tr_body, 0)
                store_copy(c, b).start()

                @pl.when(c + 2 < C)
                def _():
                    gather_copy(c + 2, b).start()

            return carry

        lax.fori_loop(0, C // 2, outer, 0)
        store_copy(C - 2, 0).wait()
        store_copy(C - 1, 1).wait()

    return k


def kernel(x, lut):
    R, C = x.shape
    V = lut.shape[0]
    rpw = R // _NW
    cb_n = C // _TR
    db_n = D_MODEL // _TR

    x2 = x.astype(jnp.int32).T
    x2 = with_layout_constraint(
        x2, Layout(major_to_minor=(0, 1), tiling=((_TR, _TC),))
    )
    x3 = x2.reshape(cb_n, _TR, _NW, rpw)
    x3 = with_layout_constraint(
        x3, Layout(major_to_minor=(0, 2, 1, 3), tiling=())
    )
    x4 = x3.transpose(0, 2, 1, 3)
    x4 = with_layout_constraint(
        x4, Layout(major_to_minor=(0, 1, 2, 3), tiling=())
    )

    lut2 = with_layout_constraint(
        lut, Layout(major_to_minor=(0, 1), tiling=())
    )
    o5 = _make_kernel(R, C, V)(x4, lut2)

    t = o5.transpose(2, 4, 0, 1, 3)
    t = with_layout_constraint(
        t, Layout(major_to_minor=(2, 3, 0, 4, 1), tiling=())
    )
    return t.reshape(R, C, D_MODEL)
